# baseline (speedup 1.0000x reference)
import functools

import jax
import jax.numpy as jnp
from jax import lax
from jax.experimental import pallas as pl
from jax.experimental.pallas import tpu as pltpu
from jax.experimental.pallas import tpu_sc as plsc

_N_NODES = 10000
_D = 128
_BATCH = 16384

_NC = 2
_NS = 16
_NW = _NC * _NS
_B_PER_W = _BATCH // _NW
_CHUNK = 128
_N_CHUNKS = _B_PER_W // _CHUNK

_ROW_BLK = 10000


def _transform_body(tab_ref, W1_ref, b1_ref, Wc_ref, bc_ref, We1_ref,
                    be1_ref, We2_ref, be2_ref, out_ref):
    t = tab_ref[...]
    g = jnp.maximum(
        jnp.dot(t, W1_ref[...], preferred_element_type=jnp.float32)
        + b1_ref[...], 0.0)
    Wf = jnp.dot(Wc_ref[...], We1_ref[...], preferred_element_type=jnp.float32)
    bf = jnp.dot(bc_ref[...], We1_ref[...],
                 preferred_element_type=jnp.float32) + be1_ref[...]
    h = jnp.maximum(
        jnp.dot(g, Wf, preferred_element_type=jnp.float32) + bf, 0.0)
    out_ref[...] = jnp.dot(
        h, We2_ref[...], preferred_element_type=jnp.float32) + be2_ref[...]


def _transform_rows(rows, W1, b1, Wc, bc, We1, be1, We2, be2, row_blk):
    n_rows = rows.shape[0]
    full = pl.BlockSpec((_D, _D), lambda i: (0, 0))
    vec = pl.BlockSpec((1, _D), lambda i: (0, 0))
    return pl.pallas_call(
        _transform_body,
        grid=(n_rows // row_blk,),
        in_specs=[
            pl.BlockSpec((row_blk, _D), lambda i: (i, 0)),
            full, vec, full, vec, full, vec, full, vec,
        ],
        out_specs=pl.BlockSpec((row_blk, _D), lambda i: (i, 0)),
        out_shape=jax.ShapeDtypeStruct((n_rows, _D), jnp.float32),
    )(rows, W1, b1.reshape(1, _D), Wc, bc.reshape(1, _D),
      We1, be1.reshape(1, _D), We2, be2.reshape(1, _D))


def _gather_body(tab_hbm, idx_hbm, out_hbm, idx_v, rows_v, sem):
    wid = lax.axis_index("s") * _NC + lax.axis_index("c")
    pltpu.sync_copy(idx_hbm.at[wid], idx_v)
    gathers = []
    for j in range(_N_CHUNKS):
        gathers.append(pltpu.async_copy(
            tab_hbm.at[idx_v.at[j]],
            rows_v.at[pl.ds(j * _CHUNK, _CHUNK), :],
            sem))
    for g in gathers:
        g.wait()
    pltpu.sync_copy(rows_v, out_hbm.at[pl.ds(wid * _B_PER_W, _B_PER_W)])


@functools.cache
def _make_sc_gather():
    return functools.partial(
        pl.kernel,
        out_type=jax.ShapeDtypeStruct((_BATCH, _D), jnp.float32),
        mesh=plsc.VectorSubcoreMesh(
            core_axis_name="c", subcore_axis_name="s",
            num_cores=_NC, num_subcores=_NS),
        scratch_types=[
            pltpu.VMEM((_N_CHUNKS, _CHUNK), jnp.int32),
            pltpu.VMEM((_B_PER_W, _D), jnp.float32),
            pltpu.SemaphoreType.DMA,
        ],
    )(_gather_body)


def kernel(table, W1, b1, Wc, bc, We1, be1, We2, be2,
           edge_weight, pert_indices, edge_index):
    del edge_weight, edge_index
    f_table = _transform_rows(table, W1, b1, Wc, bc, We1, be1, We2, be2,
                              row_blk=_ROW_BLK)
    dep = jnp.float32(0.0) * f_table[0, 0]
    idx = pert_indices.reshape(_NW, _N_CHUNKS, _CHUNK) + dep.astype(jnp.int32)
    return _make_sc_gather()(f_table, idx)


# speedup vs baseline: 1.0840x; 1.0840x over previous
import functools

import jax
import jax.numpy as jnp
from jax import lax
from jax.experimental import pallas as pl
from jax.experimental.pallas import tpu as pltpu
from jax.experimental.pallas import tpu_sc as plsc

_N_NODES = 10000
_D = 128
_BATCH = 16384

_NC = 2
_NS = 16
_NW = _NC * _NS
_B_PER_W = _BATCH // _NW
_CHUNK = 128
_N_CHUNKS = _B_PER_W // _CHUNK

_ROW_BLK = 10000


def _transform_body(tab_ref, W1_ref, b1_ref, Wc_ref, bc_ref, We1_ref,
                    be1_ref, We2_ref, be2_ref, out_ref):
    t = tab_ref[...]
    g = jnp.maximum(
        jnp.dot(t, W1_ref[...], preferred_element_type=jnp.float32)
        + b1_ref[...], 0.0)
    Wf = jnp.dot(Wc_ref[...], We1_ref[...], preferred_element_type=jnp.float32)
    bf = jnp.dot(bc_ref[...], We1_ref[...],
                 preferred_element_type=jnp.float32) + be1_ref[...]
    h = jnp.maximum(
        jnp.dot(g, Wf, preferred_element_type=jnp.float32) + bf, 0.0)
    out_ref[...] = jnp.dot(
        h, We2_ref[...], preferred_element_type=jnp.float32) + be2_ref[...]


def _transform_rows(rows, W1, b1, Wc, bc, We1, be1, We2, be2, row_blk):
    n_rows = rows.shape[0]
    full = pl.BlockSpec((_D, _D), lambda i: (0, 0))
    vec = pl.BlockSpec((1, _D), lambda i: (0, 0))
    return pl.pallas_call(
        _transform_body,
        grid=(n_rows // row_blk,),
        in_specs=[
            pl.BlockSpec((row_blk, _D), lambda i: (i, 0)),
            full, vec, full, vec, full, vec, full, vec,
        ],
        out_specs=pl.BlockSpec((row_blk, _D), lambda i: (i, 0)),
        out_shape=jax.ShapeDtypeStruct((n_rows, _D), jnp.float32),
    )(rows, W1, b1.reshape(1, _D), Wc, bc.reshape(1, _D),
      We1, be1.reshape(1, _D), We2, be2.reshape(1, _D))


def _gather_body(tab_hbm, idx_hbm, out_hbm, idx_v, rows_v, sem):
    wid = lax.axis_index("s") * _NC + lax.axis_index("c")
    pltpu.sync_copy(idx_hbm.at[wid], idx_v)
    gathers = []
    for j in range(_N_CHUNKS):
        gathers.append(pltpu.async_copy(
            tab_hbm.at[idx_v.at[j]],
            rows_v.at[pl.ds(j * _CHUNK, _CHUNK), :],
            sem))
    for g in gathers:
        g.wait()
    pltpu.sync_copy(rows_v, out_hbm.at[pl.ds(wid * _B_PER_W, _B_PER_W)])


@functools.cache
def _make_sc_gather():
    return functools.partial(
        pl.kernel,
        out_type=jax.ShapeDtypeStruct((_BATCH, _D), jnp.float32),
        mesh=plsc.VectorSubcoreMesh(
            core_axis_name="c", subcore_axis_name="s",
            num_cores=_NC, num_subcores=_NS),
        scratch_types=[
            pltpu.VMEM((_N_CHUNKS, _CHUNK), jnp.int32),
            pltpu.VMEM((_B_PER_W, _D), jnp.float32),
            pltpu.SemaphoreType.DMA,
        ],
    )(_gather_body)


def kernel(table, W1, b1, Wc, bc, We1, be1, We2, be2,
           edge_weight, pert_indices, edge_index):
    del edge_weight, edge_index
    f_table = _transform_rows(table, W1, b1, Wc, bc, We1, be1, We2, be2,
                              row_blk=_ROW_BLK)
    idx = pert_indices.reshape(_NW, _N_CHUNKS, _CHUNK)
    return _make_sc_gather()(f_table, idx)
